# baseline (speedup 1.0000x reference)
import functools

import jax
import jax.numpy as jnp
from jax import lax
from jax.experimental import pallas as pl
from jax.experimental.pallas import tpu as pltpu
from jax.experimental.pallas import tpu_sc as plsc

NUM_CORES = 2
NUM_SUBCORES = 16
NUM_WORKERS = NUM_CORES * NUM_SUBCORES

S = 4096
T = 200
B = S * T
D = 64

STREAM = 100
GS = 2
GROUP = GS * T
G = GROUP // STREAM
G2 = G // GS

S_PER_W = S // NUM_WORKERS
N_GROUPS = S_PER_W // GS
ROWS_PER_W = S_PER_W * T // STREAM

NBUF = 2

_mesh = plsc.VectorSubcoreMesh(core_axis_name="c", subcore_axis_name="s")


@functools.partial(
    pl.kernel,
    mesh=_mesh,
    out_type=jax.ShapeDtypeStruct((S, T, D), jnp.float32),
    scratch_types=[
        pltpu.VMEM((ROWS_PER_W, STREAM), jnp.int32),
        pltpu.VMEM((GS, T, 2 * D), jnp.float32),
        pltpu.VMEM((GS, T, 2 * D), jnp.float32),
        pltpu.SemaphoreType.DMA,
        pltpu.SemaphoreType.DMA,
        pltpu.SemaphoreType.DMA,
        pltpu.SemaphoreType.DMA,
    ],
    compiler_params=pltpu.CompilerParams(use_tc_tiling_on_sc=False),
)
def _emb_lookup(idx_hbm, table_hbm, out_hbm, idx_v, rows0, rows1,
                gat_sem0, gat_sem1, wb_sem0, wb_sem1):
    rows = (rows0, rows1)
    gat_sem = (gat_sem0, gat_sem1)
    wb_sem = (wb_sem0, wb_sem1)

    wid = lax.axis_index("s") * NUM_CORES + lax.axis_index("c")
    s_base = wid * S_PER_W

    pltpu.async_copy(
        idx_hbm.at[pl.ds(wid * ROWS_PER_W, ROWS_PER_W)], idx_v, gat_sem0
    ).wait()

    def streams(g, b):
        out = []
        for j in range(G):
            out.append((
                table_hbm.at[idx_v.at[g * G + j]],
                rows[b].at[j // G2, pl.ds((j % G2) * STREAM, STREAM)],
                gat_sem[b],
            ))
        return out

    def fire(g, b):
        for src, dst, sem in streams(g, b):
            pltpu.async_copy(src, dst, sem)

    def drain_gathers(g, b):
        for src, dst, sem in streams(g, b):
            pltpu.make_async_copy(src, dst, sem).wait()

    for b in range(NBUF):
        fire(b, b)

    def body(k, _):
        for b in range(NBUF):
            g_done = k * NBUF + b
            drain_gathers(g_done, b)
            pltpu.async_copy(
                rows[b].at[:, :, pl.ds(0, D)],
                out_hbm.at[pl.ds(s_base + g_done * GS, GS)],
                wb_sem[b],
            ).wait()
            fire(g_done + NBUF, b)
        return ()

    lax.fori_loop(0, N_GROUPS // NBUF - 1, body, (), unroll=False)

    for b in range(NBUF):
        g_done = N_GROUPS - NBUF + b
        drain_gathers(g_done, b)
        pltpu.async_copy(
            rows[b].at[:, :, pl.ds(0, D)],
            out_hbm.at[pl.ds(s_base + g_done * GS, GS)],
            wb_sem[b],
        ).wait()


def kernel(sentences_indices, table):
    idx2d = sentences_indices.reshape(B // STREAM, STREAM).astype(jnp.int32)
    table128 = jnp.pad(table, ((0, 0), (0, D)))
    return _emb_lookup(idx2d, table128)
